# baseline (speedup 1.0000x reference)
import functools

import jax
import jax.numpy as jnp
from jax import lax
from jax.experimental import pallas as pl
from jax.experimental.pallas import tpu as pltpu
from jax.experimental.pallas import tpu_sc as plsc

N = 10000
D = 128
NC = 2
NS = 16
NW = NC * NS
L = 16
CHUNK = 64
NBUF = 4
SEG = 32
NP = 10240
RPT = NP // NS
ZC = RPT // CHUNK

_mesh = plsc.VectorSubcoreMesh(core_axis_name="c", subcore_axis_name="s")


def _ceil_div(a, b):
    return (a + b - 1) // b


def _deg_kernel(ch):
    chg = ch // SEG

    @functools.partial(
        pl.kernel,
        out_type=jax.ShapeDtypeStruct((NC, NP), jnp.float32),
        mesh=_mesh,
        scratch_types=[
            pltpu.VMEM_SHARED((NP,), jnp.float32),
            pltpu.VMEM((2, SEG, CHUNK), jnp.int32),
            pltpu.VMEM((CHUNK,), jnp.float32),
            pltpu.VMEM((RPT,), jnp.float32),
            pltpu.SemaphoreType.DMA,
            pltpu.SemaphoreType.DMA,
            pltpu.SemaphoreType.DMA,
        ],
    )
    def k(dstm_hbm, out_hbm, dacc, dst_v, ones_v, zero_v, semi0, semi1, semd):
        cid = lax.axis_index("c")
        sid = lax.axis_index("s")
        wid = cid * NS + sid
        semi = (semi0, semi1)
        pltpu.async_copy(
            dstm_hbm.at[wid, pl.ds(0, SEG)], dst_v.at[0], semi[0])
        for j in range(CHUNK // L):
            ones_v[pl.ds(j * L, L)] = jnp.ones((L,), jnp.float32)

        def zfill(i, _):
            zero_v[pl.ds(i * L, L)] = jnp.zeros((L,), jnp.float32)
            return 0

        lax.fori_loop(0, RPT // L, zfill, 0)
        pltpu.sync_copy(zero_v, dacc.at[pl.ds(sid * RPT, RPT)])
        plsc.subcore_barrier()

        for g in range(chg):
            p = g % 2
            pltpu.make_async_copy(
                dstm_hbm.at[wid, pl.ds(g * SEG, SEG)],
                dst_v.at[p], semi[p]).wait()
            if g + 1 < chg:
                pltpu.async_copy(
                    dstm_hbm.at[wid, pl.ds((g + 1) * SEG, SEG)],
                    dst_v.at[1 - p], semi[1 - p])

            def body(j, _):
                pltpu.async_copy(ones_v, dacc.at[dst_v.at[p, j]], semd,
                                 add=True)
                return 0

            lax.fori_loop(0, SEG, body, 0)

            def drain(j, _):
                pltpu.make_async_copy(
                    ones_v, dacc.at[dst_v.at[p, 0]], semd).wait()
                return 0

            lax.fori_loop(0, SEG, drain, 0)
        plsc.subcore_barrier()
        pltpu.sync_copy(dacc.at[pl.ds(sid * RPT, RPT)],
                        out_hbm.at[cid, pl.ds(sid * RPT, RPT)])

    return k


def _rows_kernel(ch):
    assert ch % SEG == 0 and SEG % 2 == 0
    chg = ch // SEG

    @functools.partial(
        pl.kernel,
        out_type=jax.ShapeDtypeStruct((NC, NP, D), jnp.float32),
        mesh=_mesh,
        scratch_types=[
            pltpu.VMEM_SHARED((NP, D), jnp.float32),
            pltpu.VMEM((2, 2, SEG, CHUNK), jnp.int32),
            pltpu.VMEM((NBUF, CHUNK, D), jnp.float32),
            pltpu.SemaphoreType.DMA,
            pltpu.SemaphoreType.DMA,
            pltpu.SemaphoreType.DMA,
            pltpu.SemaphoreType.DMA,
            pltpu.SemaphoreType.DMA,
            pltpu.SemaphoreType.DMA,
            pltpu.SemaphoreType.DMA,
        ],
    )
    def k(table_hbm, idxm_hbm, out_hbm, acc, idx_v, rows_v,
          semi0, semi1, sg0, sg1, sg2, sg3, semz):
        cid = lax.axis_index("c")
        sid = lax.axis_index("s")
        wid = cid * NS + sid
        semi = (semi0, semi1)
        sg = (sg0, sg1, sg2, sg3)
        pltpu.async_copy(idxm_hbm.at[wid, 0], idx_v.at[0], semi[0])

        def zfill(i, _):
            for j in range(D // L):
                rows_v[0, i, pl.ds(j * L, L)] = jnp.zeros((L,), jnp.float32)
            return 0

        lax.fori_loop(0, CHUNK, zfill, 0)
        for j in range(ZC):
            pltpu.async_copy(
                rows_v.at[0], acc.at[pl.ds(sid * RPT + j * CHUNK, CHUNK)],
                semz)
        for j in range(ZC):
            pltpu.make_async_copy(
                rows_v.at[0], acc.at[pl.ds(sid * RPT, CHUNK)], semz).wait()
        pltpu.make_async_copy(
            idxm_hbm.at[wid, 0], idx_v.at[0], semi[0]).wait()
        plsc.subcore_barrier()

        for g in range(chg):
            p = g % 2
            if g + 1 < chg:
                pltpu.async_copy(
                    idxm_hbm.at[wid, g + 1], idx_v.at[1 - p], semi[1 - p])
            src_v = idx_v.at[p, 0]
            dst_v = idx_v.at[p, 1]

            for b in range(NBUF - 1):
                pltpu.async_copy(
                    table_hbm.at[src_v.at[b]], rows_v.at[b], sg[b])

            def body(i, _):
                for b in range(NBUF):
                    j = NBUF * i + b
                    pltpu.make_async_copy(
                        table_hbm.at[src_v.at[j]], rows_v.at[b], sg[b]).wait()
                    bn = (b + NBUF - 1) % NBUF

                    @pl.when(j + NBUF - 1 < SEG)
                    def _():
                        pltpu.async_copy(
                            table_hbm.at[src_v.at[j + NBUF - 1]],
                            rows_v.at[bn], sg[bn])

                    pltpu.sync_copy(
                        rows_v.at[b], acc.at[dst_v.at[j]], add=True)
                return 0

            lax.fori_loop(0, SEG // NBUF, body, 0)
            if g + 1 < chg:
                pltpu.make_async_copy(
                    idxm_hbm.at[wid, g + 1], idx_v.at[1 - p],
                    semi[1 - p]).wait()
        plsc.subcore_barrier()
        pltpu.sync_copy(acc.at[pl.ds(sid * RPT, RPT)],
                        out_hbm.at[cid, pl.ds(sid * RPT, RPT)])

    return k


_R = 1000


def _tc1a(x, W1):
    def body(x_b, w_b, xw_b):
        xw_b[...] = jnp.dot(
            x_b[...], w_b[...], preferred_element_type=jnp.float32)

    return pl.pallas_call(
        body,
        grid=(N // _R,),
        in_specs=[
            pl.BlockSpec((_R, D), lambda i: (i, 0)),
            pl.BlockSpec((D, D), lambda i: (0, 0)),
        ],
        out_specs=pl.BlockSpec((_R, D), lambda i: (i, 0)),
        out_shape=jax.ShapeDtypeStruct((N, D), jnp.float32),
    )(x, W1)


def _tc1b(xw, dga, dgb):
    def body(xw_b, da_b, db_b, dis_b, t1_b):
        dis = lax.rsqrt(da_b[...] + db_b[...] + 1.0)
        dis_b[...] = dis
        t1_b[...] = xw_b[...] * dis

    return pl.pallas_call(
        body,
        grid=(N // _R,),
        in_specs=[
            pl.BlockSpec((_R, D), lambda i: (i, 0)),
            pl.BlockSpec((_R, 1), lambda i: (i, 0)),
            pl.BlockSpec((_R, 1), lambda i: (i, 0)),
        ],
        out_specs=[
            pl.BlockSpec((_R, 1), lambda i: (i, 0)),
            pl.BlockSpec((_R, D), lambda i: (i, 0)),
        ],
        out_shape=[
            jax.ShapeDtypeStruct((N, 1), jnp.float32),
            jax.ShapeDtypeStruct((N, D), jnp.float32),
        ],
    )(xw, dga, dgb)


def _tc2(t1, accp, dis, b1, W2):
    def body(t_b, p0_b, p1_b, d_b, b_b, w_b, o_b):
        p = (p0_b[...] + p1_b[...]).reshape(_R, D)
        z = d_b[...] * (p + t_b[...]) + b_b[...]
        z = jnp.maximum(z, 0.0)
        o_b[...] = jnp.dot(
            z, w_b[...], preferred_element_type=jnp.float32) * d_b[...]

    return pl.pallas_call(
        body,
        grid=(N // _R,),
        in_specs=[
            pl.BlockSpec((_R, D), lambda i: (i, 0)),
            pl.BlockSpec((1, _R, D), lambda i: (0, i, 0)),
            pl.BlockSpec((1, _R, D), lambda i: (1, i, 0)),
            pl.BlockSpec((_R, 1), lambda i: (i, 0)),
            pl.BlockSpec((1, D), lambda i: (0, 0)),
            pl.BlockSpec((D, D), lambda i: (0, 0)),
        ],
        out_specs=pl.BlockSpec((_R, D), lambda i: (i, 0)),
        out_shape=jax.ShapeDtypeStruct((N, D), jnp.float32),
    )(t1, accp, accp, dis, b1, W2)


def _tc3(t2, accp, dis, b2):
    def body(t_b, q0_b, q1_b, d_b, b_b, o_b):
        q = (q0_b[...] + q1_b[...]).reshape(_R, D)
        o_b[...] = d_b[...] * (q + t_b[...]) + b_b[...]

    return pl.pallas_call(
        body,
        grid=(N // _R,),
        in_specs=[
            pl.BlockSpec((_R, D), lambda i: (i, 0)),
            pl.BlockSpec((1, _R, D), lambda i: (0, i, 0)),
            pl.BlockSpec((1, _R, D), lambda i: (1, i, 0)),
            pl.BlockSpec((_R, 1), lambda i: (i, 0)),
            pl.BlockSpec((1, D), lambda i: (0, 0)),
        ],
        out_specs=pl.BlockSpec((_R, D), lambda i: (i, 0)),
        out_shape=jax.ShapeDtypeStruct((N, D), jnp.float32),
    )(t2, accp, accp, dis, b2)


def kernel(x, edge_index, W1, b1, W2, b2):
    E = edge_index.shape[1]
    ch = SEG * _ceil_div(E, NW * CHUNK * SEG)
    e_pad = NW * ch * CHUNK
    pad = e_pad - E
    src = edge_index[0]
    dst = edge_index[1]
    if pad:
        ar = jnp.arange(pad, dtype=jnp.int32)
        src = jnp.concatenate([src, (ar * 997) % N])
        dst = jnp.concatenate([dst, N + (ar % (NP - N))])
    chg = ch // SEG
    dstm = dst.reshape(NW, ch, CHUNK)
    idxm = jnp.stack([src.reshape(NW, chg, SEG, CHUNK),
                      dst.reshape(NW, chg, SEG, CHUNK)], axis=2)

    degp = _deg_kernel(ch)(dstm)
    xw = _tc1a(x, W1)
    dis, t1 = _tc1b(xw, degp[0, :N, None], degp[1, :N, None])

    rows = _rows_kernel(ch)
    acc1 = rows(t1, idxm)
    t2 = _tc2(t1, acc1, dis, b1.reshape(1, D), W2)
    acc2 = rows(t2, idxm)
    return _tc3(t2, acc2, dis, b2.reshape(1, D))


# speedup vs baseline: 1.0267x; 1.0267x over previous
import functools

import jax
import jax.numpy as jnp
from jax import lax
from jax.experimental import pallas as pl
from jax.experimental.pallas import tpu as pltpu
from jax.experimental.pallas import tpu_sc as plsc

N = 10000
D = 128
NC = 2
NS = 16
NW = NC * NS
L = 16
CHUNK = 64
NBUF = 4
SEG = 32
NP = 10240
RPT = NP // NS
ZC = RPT // CHUNK

_mesh = plsc.VectorSubcoreMesh(core_axis_name="c", subcore_axis_name="s")


def _ceil_div(a, b):
    return (a + b - 1) // b


def _deg_kernel(ch):
    chg = ch // SEG

    @functools.partial(
        pl.kernel,
        out_type=jax.ShapeDtypeStruct((NC, NP), jnp.float32),
        mesh=_mesh,
        scratch_types=[
            pltpu.VMEM_SHARED((NP,), jnp.float32),
            pltpu.VMEM((2, SEG, CHUNK), jnp.int32),
            pltpu.VMEM((CHUNK,), jnp.float32),
            pltpu.VMEM((RPT,), jnp.float32),
            pltpu.SemaphoreType.DMA,
            pltpu.SemaphoreType.DMA,
            pltpu.SemaphoreType.DMA,
        ],
    )
    def k(dstm_hbm, out_hbm, dacc, dst_v, ones_v, zero_v, semi0, semi1, semd):
        cid = lax.axis_index("c")
        sid = lax.axis_index("s")
        wid = cid * NS + sid
        semi = (semi0, semi1)
        pltpu.async_copy(
            dstm_hbm.at[wid, pl.ds(0, SEG)], dst_v.at[0], semi[0])
        for j in range(CHUNK // L):
            ones_v[pl.ds(j * L, L)] = jnp.ones((L,), jnp.float32)

        def zfill(i, _):
            zero_v[pl.ds(i * L, L)] = jnp.zeros((L,), jnp.float32)
            return 0

        lax.fori_loop(0, RPT // L, zfill, 0)
        pltpu.sync_copy(zero_v, dacc.at[pl.ds(sid * RPT, RPT)])
        plsc.subcore_barrier()

        for g in range(chg):
            p = g % 2
            pltpu.make_async_copy(
                dstm_hbm.at[wid, pl.ds(g * SEG, SEG)],
                dst_v.at[p], semi[p]).wait()
            if g + 1 < chg:
                pltpu.async_copy(
                    dstm_hbm.at[wid, pl.ds((g + 1) * SEG, SEG)],
                    dst_v.at[1 - p], semi[1 - p])

            def body(j, _):
                pltpu.async_copy(ones_v, dacc.at[dst_v.at[p, j]], semd,
                                 add=True)
                return 0

            lax.fori_loop(0, SEG, body, 0)

            def drain(j, _):
                pltpu.make_async_copy(
                    ones_v, dacc.at[dst_v.at[p, 0]], semd).wait()
                return 0

            lax.fori_loop(0, SEG, drain, 0)
        plsc.subcore_barrier()
        pltpu.sync_copy(dacc.at[pl.ds(sid * RPT, RPT)],
                        out_hbm.at[cid, pl.ds(sid * RPT, RPT)])

    return k


def _rows_kernel(ch):
    assert ch % SEG == 0 and SEG % 2 == 0
    chg = ch // SEG

    @functools.partial(
        pl.kernel,
        out_type=jax.ShapeDtypeStruct((NC, NP, D), jnp.float32),
        mesh=_mesh,
        scratch_types=[
            pltpu.VMEM_SHARED((NP, D), jnp.float32),
            pltpu.VMEM((2, 2, SEG, CHUNK), jnp.int32),
            pltpu.VMEM((NBUF, CHUNK, D), jnp.float32),
            pltpu.SemaphoreType.DMA,
            pltpu.SemaphoreType.DMA,
            pltpu.SemaphoreType.DMA,
            pltpu.SemaphoreType.DMA,
            pltpu.SemaphoreType.DMA,
            pltpu.SemaphoreType.DMA,
            pltpu.SemaphoreType.DMA,
        ],
    )
    def k(table_hbm, idxm_hbm, out_hbm, acc, idx_v, rows_v,
          semi0, semi1, sg0, sg1, sg2, sg3, semz):
        cid = lax.axis_index("c")
        sid = lax.axis_index("s")
        wid = cid * NS + sid
        semi = (semi0, semi1)
        sg = (sg0, sg1, sg2, sg3)
        pltpu.async_copy(idxm_hbm.at[wid, 0], idx_v.at[0], semi[0])

        def zfill(i, _):
            for j in range(D // L):
                rows_v[0, i, pl.ds(j * L, L)] = jnp.zeros((L,), jnp.float32)
            return 0

        lax.fori_loop(0, CHUNK, zfill, 0)
        for j in range(ZC):
            pltpu.async_copy(
                rows_v.at[0], acc.at[pl.ds(sid * RPT + j * CHUNK, CHUNK)],
                semz)
        for j in range(ZC):
            pltpu.make_async_copy(
                rows_v.at[0], acc.at[pl.ds(sid * RPT, CHUNK)], semz).wait()
        pltpu.make_async_copy(
            idxm_hbm.at[wid, 0], idx_v.at[0], semi[0]).wait()
        plsc.subcore_barrier()

        for g in range(chg):
            p = g % 2
            if g + 1 < chg:
                pltpu.async_copy(
                    idxm_hbm.at[wid, g + 1], idx_v.at[1 - p], semi[1 - p])
            src_v = idx_v.at[p, 0]
            dst_v = idx_v.at[p, 1]

            for b in range(NBUF - 1):
                pltpu.async_copy(
                    table_hbm.at[src_v.at[b]], rows_v.at[b], sg[b])

            def body(i, _):
                for b in range(NBUF):
                    j = NBUF * i + b
                    pltpu.make_async_copy(
                        table_hbm.at[src_v.at[j]], rows_v.at[b], sg[b]).wait()
                    bn = (b + NBUF - 1) % NBUF

                    @pl.when(j + NBUF - 1 < SEG)
                    def _():
                        pltpu.async_copy(
                            table_hbm.at[src_v.at[j + NBUF - 1]],
                            rows_v.at[bn], sg[bn])

                    pltpu.sync_copy(
                        rows_v.at[b], acc.at[dst_v.at[j]], add=True)
                return 0

            lax.fori_loop(0, SEG // NBUF, body, 0)
            if g + 1 < chg:
                pltpu.make_async_copy(
                    idxm_hbm.at[wid, g + 1], idx_v.at[1 - p],
                    semi[1 - p]).wait()
        plsc.subcore_barrier()
        pltpu.sync_copy(acc.at[pl.ds(sid * RPT, RPT)],
                        out_hbm.at[cid, pl.ds(sid * RPT, RPT)])

    return k


_R = 1000


def _tc1(x, W1, dga, dgb):
    def body(x_b, w_b, da_b, db_b, dis_b, t1_b):
        dis = lax.rsqrt(da_b[...] + db_b[...] + 1.0)
        dis_b[...] = dis
        t1_b[...] = jnp.dot(
            x_b[...], w_b[...], preferred_element_type=jnp.float32) * dis

    return pl.pallas_call(
        body,
        grid=(N // _R,),
        in_specs=[
            pl.BlockSpec((_R, D), lambda i: (i, 0)),
            pl.BlockSpec((D, D), lambda i: (0, 0)),
            pl.BlockSpec((_R, 1), lambda i: (i, 0)),
            pl.BlockSpec((_R, 1), lambda i: (i, 0)),
        ],
        out_specs=[
            pl.BlockSpec((_R, 1), lambda i: (i, 0)),
            pl.BlockSpec((_R, D), lambda i: (i, 0)),
        ],
        out_shape=[
            jax.ShapeDtypeStruct((N, 1), jnp.float32),
            jax.ShapeDtypeStruct((N, D), jnp.float32),
        ],
    )(x, W1, dga, dgb)


def _tc2(t1, accp, dis, b1, W2):
    def body(t_b, p0_b, p1_b, d_b, b_b, w_b, o_b):
        p = (p0_b[...] + p1_b[...]).reshape(_R, D)
        z = d_b[...] * (p + t_b[...]) + b_b[...]
        z = jnp.maximum(z, 0.0)
        o_b[...] = jnp.dot(
            z, w_b[...], preferred_element_type=jnp.float32) * d_b[...]

    return pl.pallas_call(
        body,
        grid=(N // _R,),
        in_specs=[
            pl.BlockSpec((_R, D), lambda i: (i, 0)),
            pl.BlockSpec((1, _R, D), lambda i: (0, i, 0)),
            pl.BlockSpec((1, _R, D), lambda i: (1, i, 0)),
            pl.BlockSpec((_R, 1), lambda i: (i, 0)),
            pl.BlockSpec((1, D), lambda i: (0, 0)),
            pl.BlockSpec((D, D), lambda i: (0, 0)),
        ],
        out_specs=pl.BlockSpec((_R, D), lambda i: (i, 0)),
        out_shape=jax.ShapeDtypeStruct((N, D), jnp.float32),
    )(t1, accp, accp, dis, b1, W2)


def _tc3(t2, accp, dis, b2):
    def body(t_b, q0_b, q1_b, d_b, b_b, o_b):
        q = (q0_b[...] + q1_b[...]).reshape(_R, D)
        o_b[...] = d_b[...] * (q + t_b[...]) + b_b[...]

    return pl.pallas_call(
        body,
        grid=(N // _R,),
        in_specs=[
            pl.BlockSpec((_R, D), lambda i: (i, 0)),
            pl.BlockSpec((1, _R, D), lambda i: (0, i, 0)),
            pl.BlockSpec((1, _R, D), lambda i: (1, i, 0)),
            pl.BlockSpec((_R, 1), lambda i: (i, 0)),
            pl.BlockSpec((1, D), lambda i: (0, 0)),
        ],
        out_specs=pl.BlockSpec((_R, D), lambda i: (i, 0)),
        out_shape=jax.ShapeDtypeStruct((N, D), jnp.float32),
    )(t2, accp, accp, dis, b2)


def kernel(x, edge_index, W1, b1, W2, b2):
    E = edge_index.shape[1]
    ch = SEG * _ceil_div(E, NW * CHUNK * SEG)
    e_pad = NW * ch * CHUNK
    pad = e_pad - E
    src = edge_index[0]
    dst = edge_index[1]
    if pad:
        ar = jnp.arange(pad, dtype=jnp.int32)
        src = jnp.concatenate([src, (ar * 997) % N])
        dst = jnp.concatenate([dst, N + (ar % (NP - N))])
    chg = ch // SEG
    dstm = dst.reshape(NW, ch, CHUNK)
    idxm = jnp.stack([src.reshape(NW, chg, SEG, CHUNK),
                      dst.reshape(NW, chg, SEG, CHUNK)], axis=2)

    degp = _deg_kernel(ch)(dstm)
    dis, t1 = _tc1(x, W1, degp[0, :N, None], degp[1, :N, None])

    rows = _rows_kernel(ch)
    acc1 = rows(t1, idxm)
    t2 = _tc2(t1, acc1, dis, b1.reshape(1, D), W2)
    acc2 = rows(t2, idxm)
    return _tc3(t2, acc2, dis, b2.reshape(1, D))
